# Initial kernel scaffold; baseline (speedup 1.0000x reference)
import jax
import jax.numpy as jnp
from jax import lax
from jax.experimental import pallas as pl
from jax.experimental.pallas import tpu as pltpu
from jax.experimental.pallas import tpu_sc as plsc

N = 10000
POS = 3
F = 128
M = 16
H1 = 514
HC = 64
HN = 256
DX = 144
ACCW = 32
NC, NS = 2, 16
NW = NC * NS
NPAD = 10240
ZROWS = NPAD // NS
EBLK = 512
NBLK = 1000


def _sc_gather_body(epw, k, src_hbm, dst_hbm, xr_hbm, gj_hbm, gi_hbm,
                    idxj_v, idxi_v, rowsj_v, rowsi_v, semj, semi):
    wid = lax.axis_index("s") * NC + lax.axis_index("c")
    base = wid * epw

    def step(i, carry):
        off = base + i * k
        pltpu.sync_copy(src_hbm.at[pl.ds(off, k)], idxj_v)
        pltpu.sync_copy(dst_hbm.at[pl.ds(off, k)], idxi_v)
        cj = pltpu.async_copy(xr_hbm.at[idxj_v], rowsj_v, semj)
        ci = pltpu.async_copy(xr_hbm.at[idxi_v], rowsi_v, semi)
        cj.wait()
        pltpu.sync_copy(rowsj_v, gj_hbm.at[pl.ds(off, k)])
        ci.wait()
        pltpu.sync_copy(rowsi_v, gi_hbm.at[pl.ds(off, k)])
        return carry

    lax.fori_loop(0, epw // k, step, 0)


def _sc_scatter_body(epw, k, dst_hbm, ev_hbm, z_hbm, out_hbm,
                     idx_v, vals_v, acc_sh):
    cid = lax.axis_index("c")
    sid = lax.axis_index("s")
    wid = sid * NC + cid
    pltpu.sync_copy(z_hbm, acc_sh.at[pl.ds(sid * ZROWS, ZROWS)])
    plsc.subcore_barrier()
    base = wid * epw

    def step(i, carry):
        off = base + i * k
        pltpu.sync_copy(dst_hbm.at[pl.ds(off, k)], idx_v)
        pltpu.sync_copy(ev_hbm.at[pl.ds(off, k)], vals_v)
        pltpu.sync_copy(vals_v, acc_sh.at[idx_v], add=True)
        return carry

    lax.fori_loop(0, epw // k, step, 0)
    plsc.subcore_barrier()
    pltpu.sync_copy(acc_sh.at[pl.ds(sid * ZROWS, ZROWS)],
                    out_hbm.at[cid, pl.ds(sid * ZROWS, ZROWS)])


def _edge_mlp_kernel(gi_ref, gj_ref, w1a_ref, w1b_ref, w1r_ref, b1_ref,
                     w2_ref, b2_ref, wc1_ref, bc1_ref, wc2_ref, bc2_ref,
                     cs_ref, out_ref):
    gi = gi_ref[...]
    gj = gj_ref[...]
    rel = gj[:, F:F + 8] - gi[:, F:F + 8]
    rel_dist = jnp.sum(rel * rel, axis=1, keepdims=True)
    h = (jnp.dot(gi, w1a_ref[...], preferred_element_type=jnp.float32)
         + jnp.dot(gj, w1b_ref[...], preferred_element_type=jnp.float32)
         + rel_dist * w1r_ref[...] + b1_ref[...])
    h = h * jax.nn.sigmoid(h)
    m = jnp.dot(h, w2_ref[...], preferred_element_type=jnp.float32) + b2_ref[...]
    m = m * jax.nn.sigmoid(m)
    c = jnp.dot(m, wc1_ref[...], preferred_element_type=jnp.float32) + bc1_ref[...]
    c = c * jax.nn.sigmoid(c)
    w = jnp.tanh(jnp.sum(c * wc2_ref[...], axis=1, keepdims=True) + bc2_ref[...])
    inv = w * cs_ref[...] / jnp.maximum(jnp.sqrt(rel_dist), 1e-8)
    wvec = rel * inv
    out_ref[...] = jnp.concatenate([m, wvec, jnp.zeros_like(wvec)], axis=1)


def _node_mlp_kernel(x_ref, a0_ref, a1_ref, wn1a_ref, wn1b_ref, bn1_ref,
                     wn2_ref, bn2_ref, out_ref):
    xv = x_ref[...]
    feats = xv[:, POS:]
    a = a0_ref[...] + a1_ref[...]
    m_i = a[:, :M]
    mhat = a[:, M:M + POS]
    h = (jnp.dot(feats, wn1a_ref[...], preferred_element_type=jnp.float32)
         + jnp.dot(m_i, wn1b_ref[...], preferred_element_type=jnp.float32)
         + bn1_ref[...])
    h = h * jax.nn.sigmoid(h)
    hid = jnp.dot(h, wn2_ref[...], preferred_element_type=jnp.float32) + bn2_ref[...]
    out_ref[...] = jnp.concatenate([xv[:, :POS] + mhat, feats + hid], axis=1)


def kernel(x, edge_index, W1, b1, W2, b2, Wc1, bc1, Wc2, bc2,
           Wn1, bn1, Wn2, bn2, coors_scale):
    E = edge_index.shape[1]
    epw = E // NW
    k = 400
    assert E % NW == 0 and epw % k == 0 and k % 8 == 0

    f32 = jnp.float32
    xr = jnp.concatenate(
        [x[:, POS:], x[:, :POS], jnp.zeros((N, DX - POS - F), f32)], axis=1)
    src = edge_index[0]
    dst = edge_index[1]

    mesh = plsc.VectorSubcoreMesh(core_axis_name="c", subcore_axis_name="s")

    gj, gi = pl.kernel(
        lambda *refs: _sc_gather_body(epw, k, *refs),
        out_type=[jax.ShapeDtypeStruct((E, DX), f32),
                  jax.ShapeDtypeStruct((E, DX), f32)],
        mesh=mesh,
        scratch_types=[pltpu.VMEM((k,), jnp.int32),
                       pltpu.VMEM((k,), jnp.int32),
                       pltpu.VMEM((k, DX), f32),
                       pltpu.VMEM((k, DX), f32),
                       pltpu.SemaphoreType.DMA,
                       pltpu.SemaphoreType.DMA],
    )(src, dst, xr)

    zpad = jnp.zeros((DX - F, H1), f32)
    w1a = jnp.concatenate([W1[:F], zpad], axis=0)
    w1b = jnp.concatenate([W1[F:2 * F], zpad], axis=0)
    w1r = W1[2 * F:2 * F + 1]

    ev = pl.pallas_call(
        _edge_mlp_kernel,
        grid=(E // EBLK,),
        in_specs=[
            pl.BlockSpec((EBLK, DX), lambda i: (i, 0)),
            pl.BlockSpec((EBLK, DX), lambda i: (i, 0)),
            pl.BlockSpec((DX, H1), lambda i: (0, 0)),
            pl.BlockSpec((DX, H1), lambda i: (0, 0)),
            pl.BlockSpec((1, H1), lambda i: (0, 0)),
            pl.BlockSpec((1, H1), lambda i: (0, 0)),
            pl.BlockSpec((H1, M), lambda i: (0, 0)),
            pl.BlockSpec((1, M), lambda i: (0, 0)),
            pl.BlockSpec((M, HC), lambda i: (0, 0)),
            pl.BlockSpec((1, HC), lambda i: (0, 0)),
            pl.BlockSpec((1, HC), lambda i: (0, 0)),
            pl.BlockSpec((1, 1), lambda i: (0, 0)),
            pl.BlockSpec((1, 1), lambda i: (0, 0)),
        ],
        out_specs=pl.BlockSpec((EBLK, ACCW), lambda i: (i, 0)),
        out_shape=jax.ShapeDtypeStruct((E, ACCW), f32),
    )(gi, gj, w1a, w1b, w1r, b1[None], W2, b2[None], Wc1, bc1[None],
      Wc2[:, 0][None], bc2[None], coors_scale[None])

    zrows = jnp.zeros((ZROWS, ACCW), f32)
    parts = pl.kernel(
        lambda *refs: _sc_scatter_body(epw, k, *refs),
        out_type=jax.ShapeDtypeStruct((NC, NPAD, ACCW), f32),
        mesh=mesh,
        scratch_types=[pltpu.VMEM((k,), jnp.int32),
                       pltpu.VMEM((k, ACCW), f32),
                       pltpu.VMEM_SHARED((NPAD, ACCW), f32)],
    )(dst, ev, zrows)

    out = pl.pallas_call(
        _node_mlp_kernel,
        grid=(N // NBLK,),
        in_specs=[
            pl.BlockSpec((NBLK, POS + F), lambda i: (i, 0)),
            pl.BlockSpec((NBLK, ACCW), lambda i: (i, 0)),
            pl.BlockSpec((NBLK, ACCW), lambda i: (i, 0)),
            pl.BlockSpec((F, HN), lambda i: (0, 0)),
            pl.BlockSpec((M, HN), lambda i: (0, 0)),
            pl.BlockSpec((1, HN), lambda i: (0, 0)),
            pl.BlockSpec((HN, F), lambda i: (0, 0)),
            pl.BlockSpec((1, F), lambda i: (0, 0)),
        ],
        out_specs=pl.BlockSpec((NBLK, POS + F), lambda i: (i, 0)),
        out_shape=jax.ShapeDtypeStruct((N, POS + F), f32),
    )(x, parts[0, :N], parts[1, :N], Wn1[:F], Wn1[F:], bn1[None],
      Wn2, bn2[None])
    return out


# speedup vs baseline: 2.5393x; 2.5393x over previous
import jax
import jax.numpy as jnp
from jax import lax
from jax.experimental import pallas as pl
from jax.experimental.pallas import tpu as pltpu
from jax.experimental.pallas import tpu_sc as plsc

N = 10000
POS = 3
F = 128
M = 16
H1 = 514
HC = 64
HN = 256
DX = 128
ACCW = 128
NC, NS = 2, 16
NW = NC * NS
NPAD = 10240
ZROWS = NPAD // NS
EBLK = 512
NBLK = 1000


def _sc_gather_body(epw, k, src_hbm, dst_hbm, xr_hbm, gj_hbm, gi_hbm,
                    idxj_v, idxi_v, rowsj_v, rowsi_v, semj, semi):
    wid = lax.axis_index("s") * NC + lax.axis_index("c")
    base = wid * epw

    def step(i, carry):
        off = base + i * k
        pltpu.sync_copy(src_hbm.at[pl.ds(off, k)], idxj_v)
        pltpu.sync_copy(dst_hbm.at[pl.ds(off, k)], idxi_v)
        cj = pltpu.async_copy(xr_hbm.at[idxj_v], rowsj_v, semj)
        ci = pltpu.async_copy(xr_hbm.at[idxi_v], rowsi_v, semi)
        cj.wait()
        pltpu.sync_copy(rowsj_v, gj_hbm.at[pl.ds(off, k)])
        ci.wait()
        pltpu.sync_copy(rowsi_v, gi_hbm.at[pl.ds(off, k)])
        return carry

    lax.fori_loop(0, epw // k, step, 0)


def _sc_scatter_body(epw, k, dst_hbm, ev_hbm, z_hbm, out_hbm,
                     idx_v, vals_v, acc_sh):
    cid = lax.axis_index("c")
    sid = lax.axis_index("s")
    wid = sid * NC + cid
    pltpu.sync_copy(z_hbm, acc_sh.at[pl.ds(sid * ZROWS, ZROWS)])
    plsc.subcore_barrier()
    base = wid * epw

    def step(i, carry):
        off = base + i * k
        pltpu.sync_copy(dst_hbm.at[pl.ds(off, k)], idx_v)
        pltpu.sync_copy(ev_hbm.at[pl.ds(off, k)], vals_v)
        pltpu.sync_copy(vals_v, acc_sh.at[idx_v], add=True)
        return carry

    lax.fori_loop(0, epw // k, step, 0)
    plsc.subcore_barrier()
    pltpu.sync_copy(acc_sh.at[pl.ds(sid * ZROWS, ZROWS)],
                    out_hbm.at[cid, pl.ds(sid * ZROWS, ZROWS)])


def _unpack_feats(g):
    u = jax.lax.bitcast_convert_type(g[:, :F // 2], jnp.uint32)
    fe = jax.lax.bitcast_convert_type(u << 16, jnp.float32)
    fo = jax.lax.bitcast_convert_type(u & jnp.uint32(0xFFFF0000), jnp.float32)
    return jnp.concatenate([fe, fo], axis=1).astype(jnp.bfloat16)


def _edge_mlp_kernel(gi_ref, gj_ref, w1a_ref, w1b_ref, w1r_ref, b1_ref,
                     w2_ref, b2_ref, wc1_ref, bc1_ref, wc2_ref, bc2_ref,
                     cs_ref, out_ref):
    gi = gi_ref[...]
    gj = gj_ref[...]
    fi = _unpack_feats(gi)
    fj = _unpack_feats(gj)
    rel = gj[:, F // 2:F // 2 + 8] - gi[:, F // 2:F // 2 + 8]
    rel_dist = jnp.sum(rel * rel, axis=1, keepdims=True)
    h = (jnp.dot(fi, w1a_ref[...], preferred_element_type=jnp.float32)
         + jnp.dot(fj, w1b_ref[...], preferred_element_type=jnp.float32)
         + rel_dist * w1r_ref[...] + b1_ref[...])
    h = h * jax.nn.sigmoid(h)
    m = (jnp.dot(h.astype(jnp.bfloat16), w2_ref[...],
                 preferred_element_type=jnp.float32) + b2_ref[...])
    m = m * jax.nn.sigmoid(m)
    c = jnp.dot(m, wc1_ref[...], preferred_element_type=jnp.float32) + bc1_ref[...]
    c = c * jax.nn.sigmoid(c)
    w = jnp.tanh(jnp.sum(c * wc2_ref[...], axis=1, keepdims=True) + bc2_ref[...])
    inv = w * cs_ref[...] / jnp.maximum(jnp.sqrt(rel_dist), 1e-8)
    wvec = rel * inv
    pad = jnp.zeros((wvec.shape[0], ACCW - M - 8), jnp.float32)
    out_ref[...] = jnp.concatenate([m, wvec, pad], axis=1)


def _node_mlp_kernel(x_ref, a0_ref, a1_ref, wn1a_ref, wn1b_ref, bn1_ref,
                     wn2_ref, bn2_ref, out_ref):
    xv = x_ref[...]
    feats = xv[:, POS:]
    a = a0_ref[...] + a1_ref[...]
    m_i = a[:, :M]
    mhat = a[:, M:M + POS]
    h = (jnp.dot(feats, wn1a_ref[...], preferred_element_type=jnp.float32)
         + jnp.dot(m_i, wn1b_ref[...], preferred_element_type=jnp.float32)
         + bn1_ref[...])
    h = h * jax.nn.sigmoid(h)
    hid = jnp.dot(h, wn2_ref[...], preferred_element_type=jnp.float32) + bn2_ref[...]
    out_ref[...] = jnp.concatenate([xv[:, :POS] + mhat, feats + hid], axis=1)


def kernel(x, edge_index, W1, b1, W2, b2, Wc1, bc1, Wc2, bc2,
           Wn1, bn1, Wn2, bn2, coors_scale):
    E = edge_index.shape[1]
    epw = E // NW
    k = 400
    ks = 200
    assert E % NW == 0 and epw % k == 0 and k % 8 == 0 and epw % ks == 0

    f32 = jnp.float32
    bf16 = jnp.bfloat16
    fbits = jax.lax.bitcast_convert_type(
        x[:, POS:].astype(bf16), jnp.uint16).astype(jnp.uint32)
    packed = jax.lax.bitcast_convert_type(
        fbits[:, 0::2] | (fbits[:, 1::2] << 16), f32)
    xr = jnp.concatenate(
        [packed, x[:, :POS], jnp.zeros((N, DX - POS - F // 2), f32)], axis=1)
    src = edge_index[0]
    dst = edge_index[1]

    mesh = plsc.VectorSubcoreMesh(core_axis_name="c", subcore_axis_name="s")

    gj, gi = pl.kernel(
        lambda *refs: _sc_gather_body(epw, k, *refs),
        out_type=[jax.ShapeDtypeStruct((E, DX), f32),
                  jax.ShapeDtypeStruct((E, DX), f32)],
        mesh=mesh,
        scratch_types=[pltpu.VMEM((k,), jnp.int32),
                       pltpu.VMEM((k,), jnp.int32),
                       pltpu.VMEM((k, DX), f32),
                       pltpu.VMEM((k, DX), f32),
                       pltpu.SemaphoreType.DMA,
                       pltpu.SemaphoreType.DMA],
    )(src, dst, xr)

    w1a = jnp.concatenate([W1[:F:2], W1[1:F:2]], axis=0).astype(bf16)
    w1b = jnp.concatenate([W1[F:2 * F:2], W1[F + 1:2 * F:2]], axis=0).astype(bf16)
    w1r = W1[2 * F:2 * F + 1]

    ev = pl.pallas_call(
        _edge_mlp_kernel,
        grid=(E // EBLK,),
        in_specs=[
            pl.BlockSpec((EBLK, DX), lambda i: (i, 0)),
            pl.BlockSpec((EBLK, DX), lambda i: (i, 0)),
            pl.BlockSpec((F, H1), lambda i: (0, 0)),
            pl.BlockSpec((F, H1), lambda i: (0, 0)),
            pl.BlockSpec((1, H1), lambda i: (0, 0)),
            pl.BlockSpec((1, H1), lambda i: (0, 0)),
            pl.BlockSpec((H1, M), lambda i: (0, 0)),
            pl.BlockSpec((1, M), lambda i: (0, 0)),
            pl.BlockSpec((M, HC), lambda i: (0, 0)),
            pl.BlockSpec((1, HC), lambda i: (0, 0)),
            pl.BlockSpec((1, HC), lambda i: (0, 0)),
            pl.BlockSpec((1, 1), lambda i: (0, 0)),
            pl.BlockSpec((1, 1), lambda i: (0, 0)),
        ],
        out_specs=pl.BlockSpec((EBLK, ACCW), lambda i: (i, 0)),
        out_shape=jax.ShapeDtypeStruct((E, ACCW), f32),
    )(gi, gj, w1a, w1b, w1r, b1[None], W2.astype(bf16), b2[None], Wc1, bc1[None],
      Wc2[:, 0][None], bc2[None], coors_scale[None])

    zrows = jnp.zeros((ZROWS, ACCW), f32)
    parts = pl.kernel(
        lambda *refs: _sc_scatter_body(epw, ks, *refs),
        out_type=jax.ShapeDtypeStruct((NC, NPAD, ACCW), f32),
        mesh=mesh,
        scratch_types=[pltpu.VMEM((ks,), jnp.int32),
                       pltpu.VMEM((ks, ACCW), f32),
                       pltpu.VMEM_SHARED((NPAD, ACCW), f32)],
    )(dst, ev, zrows)

    out = pl.pallas_call(
        _node_mlp_kernel,
        grid=(N // NBLK,),
        in_specs=[
            pl.BlockSpec((NBLK, POS + F), lambda i: (i, 0)),
            pl.BlockSpec((NBLK, ACCW), lambda i: (i, 0)),
            pl.BlockSpec((NBLK, ACCW), lambda i: (i, 0)),
            pl.BlockSpec((F, HN), lambda i: (0, 0)),
            pl.BlockSpec((M, HN), lambda i: (0, 0)),
            pl.BlockSpec((1, HN), lambda i: (0, 0)),
            pl.BlockSpec((HN, F), lambda i: (0, 0)),
            pl.BlockSpec((1, F), lambda i: (0, 0)),
        ],
        out_specs=pl.BlockSpec((NBLK, POS + F), lambda i: (i, 0)),
        out_shape=jax.ShapeDtypeStruct((N, POS + F), f32),
    )(x, parts[0, :N], parts[1, :N], Wn1[:F], Wn1[F:], bn1[None],
      Wn2, bn2[None])
    return out


# speedup vs baseline: 3.0159x; 1.1877x over previous
import jax
import jax.numpy as jnp
from jax import lax
from jax.experimental import pallas as pl
from jax.experimental.pallas import tpu as pltpu
from jax.experimental.pallas import tpu_sc as plsc

N = 10000
POS = 3
F = 128
M = 16
H1 = 514
HC = 64
HN = 256
DX = 128
ACCW = 128
NC, NS = 2, 16
NW = NC * NS
NPAD = 10240
ZROWS = NPAD // NS
EBLK = 640
NBLK = 1000


def _sc_gather_body(epw, k, src_hbm, dst_hbm, xr_hbm, gj_hbm, gi_hbm,
                    idxj_v, idxi_v, rowsj_v, rowsi_v, semj, semi):
    wid = lax.axis_index("s") * NC + lax.axis_index("c")
    base = wid * epw

    def step(i, carry):
        off = base + i * k
        pltpu.sync_copy(src_hbm.at[pl.ds(off, k)], idxj_v)
        pltpu.sync_copy(dst_hbm.at[pl.ds(off, k)], idxi_v)
        cj = pltpu.async_copy(xr_hbm.at[idxj_v], rowsj_v, semj)
        ci = pltpu.async_copy(xr_hbm.at[idxi_v], rowsi_v, semi)
        cj.wait()
        pltpu.sync_copy(rowsj_v, gj_hbm.at[pl.ds(off, k)])
        ci.wait()
        pltpu.sync_copy(rowsi_v, gi_hbm.at[pl.ds(off, k)])
        return carry

    lax.fori_loop(0, epw // k, step, 0)


def _sc_scatter_body(epw, k, dst_hbm, ev_hbm, z_hbm, out_hbm,
                     idx_v, vals_v, acc_sh):
    cid = lax.axis_index("c")
    sid = lax.axis_index("s")
    wid = sid * NC + cid
    pltpu.sync_copy(z_hbm, acc_sh.at[pl.ds(sid * ZROWS, ZROWS)])
    plsc.subcore_barrier()
    base = wid * epw

    def step(i, carry):
        off = base + i * k
        pltpu.sync_copy(dst_hbm.at[pl.ds(off, k)], idx_v)
        pltpu.sync_copy(ev_hbm.at[pl.ds(off, k)], vals_v)
        pltpu.sync_copy(vals_v, acc_sh.at[idx_v], add=True)
        return carry

    lax.fori_loop(0, epw // k, step, 0)
    plsc.subcore_barrier()
    pltpu.sync_copy(acc_sh.at[pl.ds(sid * ZROWS, ZROWS)],
                    out_hbm.at[cid, pl.ds(sid * ZROWS, ZROWS)])


def _silu(v):
    return v / (1.0 + jnp.exp(-v))


def _edge_mlp_kernel(gi_ref, gj_ref, w1cat_ref, w1r_ref, b1_ref,
                     w2_ref, b2_ref, wc1_ref, bc1_ref, wc2_ref, bc2_ref,
                     cs_ref, out_ref):
    gi = gi_ref[...]
    gj = gj_ref[...]
    rel = gj[:, F // 2:F // 2 + 8] - gi[:, F // 2:F // 2 + 8]
    rel_dist = jnp.sum(rel * rel, axis=1, keepdims=True)
    u = jnp.concatenate(
        [jax.lax.bitcast_convert_type(gi[:, :F // 2], jnp.uint32),
         jax.lax.bitcast_convert_type(gj[:, :F // 2], jnp.uint32)], axis=1)
    fe = jax.lax.bitcast_convert_type(u << 16, jnp.float32)
    fo = jax.lax.bitcast_convert_type(u & jnp.uint32(0xFFFF0000), jnp.float32)
    fcat = jnp.concatenate([fe, fo], axis=1).astype(jnp.bfloat16)
    rd_bf = rel_dist.astype(jnp.bfloat16)
    ha = (jnp.dot(fcat, w1cat_ref[:, :512],
                  preferred_element_type=jnp.float32).astype(jnp.bfloat16)
          + (rd_bf * w1r_ref[:, :512] + b1_ref[:, :512]))
    hb = (jnp.dot(fcat, w1cat_ref[:, 512:],
                  preferred_element_type=jnp.float32).astype(jnp.bfloat16)
          + (rd_bf * w1r_ref[:, 512:] + b1_ref[:, 512:]))
    m = (jnp.dot(_silu(ha), w2_ref[:512], preferred_element_type=jnp.float32)
         + jnp.dot(_silu(hb), w2_ref[512:], preferred_element_type=jnp.float32)
         + b2_ref[...])
    m = _silu(m)
    c = jnp.dot(m, wc1_ref[...], preferred_element_type=jnp.float32) + bc1_ref[...]
    c = _silu(c)
    w = jnp.tanh(jnp.sum(c * wc2_ref[...], axis=1, keepdims=True) + bc2_ref[...])
    inv = w * cs_ref[...] / jnp.maximum(jnp.sqrt(rel_dist), 1e-8)
    wvec = rel * inv
    pad = jnp.zeros((wvec.shape[0], ACCW - M - 8), jnp.float32)
    out_ref[...] = jnp.concatenate([m, wvec, pad], axis=1)


def _node_mlp_kernel(x_ref, a0_ref, a1_ref, wn1a_ref, wn1b_ref, bn1_ref,
                     wn2_ref, bn2_ref, out_ref):
    xv = x_ref[...]
    feats = xv[:, POS:]
    a = a0_ref[...] + a1_ref[...]
    m_i = a[:, :M]
    mhat = a[:, M:M + POS]
    h = (jnp.dot(feats, wn1a_ref[...], preferred_element_type=jnp.float32)
         + jnp.dot(m_i, wn1b_ref[...], preferred_element_type=jnp.float32)
         + bn1_ref[...])
    h = h * jax.nn.sigmoid(h)
    hid = jnp.dot(h, wn2_ref[...], preferred_element_type=jnp.float32) + bn2_ref[...]
    out_ref[...] = jnp.concatenate([xv[:, :POS] + mhat, feats + hid], axis=1)


def kernel(x, edge_index, W1, b1, W2, b2, Wc1, bc1, Wc2, bc2,
           Wn1, bn1, Wn2, bn2, coors_scale):
    E = edge_index.shape[1]
    epw = E // NW
    k = 400
    ks = 200
    assert E % NW == 0 and epw % k == 0 and k % 8 == 0 and epw % ks == 0

    f32 = jnp.float32
    bf16 = jnp.bfloat16
    fbits = jax.lax.bitcast_convert_type(
        x[:, POS:].astype(bf16), jnp.uint16).astype(jnp.uint32)
    packed = jax.lax.bitcast_convert_type(
        fbits[:, 0::2] | (fbits[:, 1::2] << 16), f32)
    xr = jnp.concatenate(
        [packed, x[:, :POS], jnp.zeros((N, DX - POS - F // 2), f32)], axis=1)
    src = edge_index[0]
    dst = edge_index[1]

    mesh = plsc.VectorSubcoreMesh(core_axis_name="c", subcore_axis_name="s")

    gj, gi = pl.kernel(
        lambda *refs: _sc_gather_body(epw, k, *refs),
        out_type=[jax.ShapeDtypeStruct((E, DX), f32),
                  jax.ShapeDtypeStruct((E, DX), f32)],
        mesh=mesh,
        scratch_types=[pltpu.VMEM((k,), jnp.int32),
                       pltpu.VMEM((k,), jnp.int32),
                       pltpu.VMEM((k, DX), f32),
                       pltpu.VMEM((k, DX), f32),
                       pltpu.SemaphoreType.DMA,
                       pltpu.SemaphoreType.DMA],
    )(src, dst, xr)

    w1cat = jnp.concatenate(
        [W1[:F:2], W1[F:2 * F:2], W1[1:F:2], W1[F + 1:2 * F:2]],
        axis=0).astype(bf16)
    w1r = W1[2 * F:2 * F + 1].astype(bf16)

    ev = pl.pallas_call(
        _edge_mlp_kernel,
        grid=(E // EBLK,),
        in_specs=[
            pl.BlockSpec((EBLK, DX), lambda i: (i, 0)),
            pl.BlockSpec((EBLK, DX), lambda i: (i, 0)),
            pl.BlockSpec((2 * F, H1), lambda i: (0, 0)),
            pl.BlockSpec((1, H1), lambda i: (0, 0)),
            pl.BlockSpec((1, H1), lambda i: (0, 0)),
            pl.BlockSpec((H1, M), lambda i: (0, 0)),
            pl.BlockSpec((1, M), lambda i: (0, 0)),
            pl.BlockSpec((M, HC), lambda i: (0, 0)),
            pl.BlockSpec((1, HC), lambda i: (0, 0)),
            pl.BlockSpec((1, HC), lambda i: (0, 0)),
            pl.BlockSpec((1, 1), lambda i: (0, 0)),
            pl.BlockSpec((1, 1), lambda i: (0, 0)),
        ],
        out_specs=pl.BlockSpec((EBLK, ACCW), lambda i: (i, 0)),
        out_shape=jax.ShapeDtypeStruct((E, ACCW), f32),
    )(gi, gj, w1cat, w1r, b1[None].astype(bf16), W2.astype(bf16), b2[None],
      Wc1, bc1[None],
      Wc2[:, 0][None], bc2[None], coors_scale[None])

    zrows = jnp.zeros((ZROWS, ACCW), f32)
    parts = pl.kernel(
        lambda *refs: _sc_scatter_body(epw, ks, *refs),
        out_type=jax.ShapeDtypeStruct((NC, NPAD, ACCW), f32),
        mesh=mesh,
        scratch_types=[pltpu.VMEM((ks,), jnp.int32),
                       pltpu.VMEM((ks, ACCW), f32),
                       pltpu.VMEM_SHARED((NPAD, ACCW), f32)],
    )(dst, ev, zrows)

    out = pl.pallas_call(
        _node_mlp_kernel,
        grid=(N // NBLK,),
        in_specs=[
            pl.BlockSpec((NBLK, POS + F), lambda i: (i, 0)),
            pl.BlockSpec((NBLK, ACCW), lambda i: (i, 0)),
            pl.BlockSpec((NBLK, ACCW), lambda i: (i, 0)),
            pl.BlockSpec((F, HN), lambda i: (0, 0)),
            pl.BlockSpec((M, HN), lambda i: (0, 0)),
            pl.BlockSpec((1, HN), lambda i: (0, 0)),
            pl.BlockSpec((HN, F), lambda i: (0, 0)),
            pl.BlockSpec((1, F), lambda i: (0, 0)),
        ],
        out_specs=pl.BlockSpec((NBLK, POS + F), lambda i: (i, 0)),
        out_shape=jax.ShapeDtypeStruct((N, POS + F), f32),
    )(x, parts[0, :N], parts[1, :N], Wn1[:F], Wn1[F:], bn1[None],
      Wn2, bn2[None])
    return out


# speedup vs baseline: 3.4029x; 1.1283x over previous
import jax
import jax.numpy as jnp
from jax import lax
from jax.experimental import pallas as pl
from jax.experimental.pallas import tpu as pltpu
from jax.experimental.pallas import tpu_sc as plsc

N = 10000
POS = 3
F = 128
M = 16
H1 = 514
HC = 64
HN = 256
DX = 128
ACCW = 128
NC, NS = 2, 16
NW = NC * NS
NPAD = 10240
ZROWS = NPAD // NS
EBLK = 640
NBLK = 1000


def _sc_gather_body(epw, k, src_hbm, dst_hbm, xr_hbm, gj_hbm, gi_hbm,
                    idxj_v, idxi_v, rowsj_v, rowsi_v, semj, semi):
    wid = lax.axis_index("s") * NC + lax.axis_index("c")
    base = wid * epw

    def step(i, carry):
        off = base + i * k
        pltpu.sync_copy(src_hbm.at[pl.ds(off, k)], idxj_v)
        pltpu.sync_copy(dst_hbm.at[pl.ds(off, k)], idxi_v)
        cj = pltpu.async_copy(xr_hbm.at[idxj_v], rowsj_v, semj)
        ci = pltpu.async_copy(xr_hbm.at[idxi_v], rowsi_v, semi)
        cj.wait()
        pltpu.sync_copy(rowsj_v, gj_hbm.at[pl.ds(off, k)])
        ci.wait()
        pltpu.sync_copy(rowsi_v, gi_hbm.at[pl.ds(off, k)])
        return carry

    lax.fori_loop(0, epw // k, step, 0)


def _sc_scatter_body(epw, k, dst_hbm, ev_hbm, z_hbm, out_hbm,
                     idx_v, vals_v, acc_sh):
    cid = lax.axis_index("c")
    sid = lax.axis_index("s")
    wid = sid * NC + cid
    pltpu.sync_copy(z_hbm, acc_sh.at[pl.ds(sid * ZROWS, ZROWS)])
    plsc.subcore_barrier()
    base = wid * epw

    def step(i, carry):
        off = base + i * k
        pltpu.sync_copy(dst_hbm.at[pl.ds(off, k)], idx_v)
        pltpu.sync_copy(ev_hbm.at[pl.ds(off, k)], vals_v)
        pltpu.sync_copy(vals_v, acc_sh.at[idx_v], add=True)
        return carry

    lax.fori_loop(0, epw // k, step, 0)
    plsc.subcore_barrier()
    pltpu.sync_copy(acc_sh.at[pl.ds(sid * ZROWS, ZROWS)],
                    out_hbm.at[cid, pl.ds(sid * ZROWS, ZROWS)])


def _silu(v):
    return v / (1.0 + jnp.exp(-v))


def _edge_mlp_kernel(gi_ref, gj_ref, w1cat_ref, w1r_ref, b1_ref,
                     w2_ref, b2_ref, wc1_ref, bc1_ref, wc2_ref, bc2_ref,
                     cs_ref, out_ref):
    gi = gi_ref[...]
    gj = gj_ref[...]
    rel = gj[:, F // 2:F // 2 + 8] - gi[:, F // 2:F // 2 + 8]
    rel_dist = jnp.sum(rel * rel, axis=1, keepdims=True)
    u = jnp.concatenate(
        [jax.lax.bitcast_convert_type(gi[:, :F // 2], jnp.uint32),
         jax.lax.bitcast_convert_type(gj[:, :F // 2], jnp.uint32)], axis=1)
    fe = jax.lax.bitcast_convert_type(u << 16, jnp.float32)
    fo = jax.lax.bitcast_convert_type(u & jnp.uint32(0xFFFF0000), jnp.float32)
    fcat = jnp.concatenate([fe, fo], axis=1).astype(jnp.bfloat16)
    rd_bf = rel_dist.astype(jnp.bfloat16)
    ha = (jnp.dot(fcat, w1cat_ref[:, :512],
                  preferred_element_type=jnp.float32).astype(jnp.bfloat16)
          + (rd_bf * w1r_ref[:, :512] + b1_ref[:, :512]))
    hb = (jnp.dot(fcat, w1cat_ref[:, 512:],
                  preferred_element_type=jnp.float32).astype(jnp.bfloat16)
          + (rd_bf * w1r_ref[:, 512:] + b1_ref[:, 512:]))
    m = (jnp.dot(_silu(ha), w2_ref[:512], preferred_element_type=jnp.float32)
         + jnp.dot(_silu(hb), w2_ref[512:], preferred_element_type=jnp.float32)
         + b2_ref[...])
    m = _silu(m)
    mb = m.astype(jnp.bfloat16)
    c = (jnp.dot(mb, wc1_ref[...], preferred_element_type=jnp.float32)
         + bc1_ref[...]).astype(jnp.bfloat16)
    c = _silu(c)
    w = jnp.tanh(jnp.sum(c * wc2_ref[...], axis=1, keepdims=True)
                 + bc2_ref[...]).astype(jnp.float32)
    inv = (w * cs_ref[...]) * jax.lax.rsqrt(jnp.maximum(rel_dist, 1e-16))
    wvec = rel * inv
    pad = jnp.zeros((wvec.shape[0], ACCW - M - 8), jnp.float32)
    out_ref[...] = jnp.concatenate([m, wvec, pad], axis=1)


def _node_mlp_kernel(x_ref, a0_ref, a1_ref, a2_ref, a3_ref,
                     wn1a_ref, wn1b_ref, bn1_ref,
                     wn2_ref, bn2_ref, out_ref):
    xv = x_ref[...]
    feats = xv[:, POS:]
    a = (a0_ref[...] + a1_ref[...]) + (a2_ref[...] + a3_ref[...])
    m_i = a[:, :M]
    mhat = a[:, M:M + POS]
    h = (jnp.dot(feats, wn1a_ref[...], preferred_element_type=jnp.float32)
         + jnp.dot(m_i, wn1b_ref[...], preferred_element_type=jnp.float32)
         + bn1_ref[...])
    h = h * jax.nn.sigmoid(h)
    hid = jnp.dot(h, wn2_ref[...], preferred_element_type=jnp.float32) + bn2_ref[...]
    out_ref[...] = jnp.concatenate([xv[:, :POS] + mhat, feats + hid], axis=1)


def kernel(x, edge_index, W1, b1, W2, b2, Wc1, bc1, Wc2, bc2,
           Wn1, bn1, Wn2, bn2, coors_scale):
    E = edge_index.shape[1]
    nh = 2
    Eh = E // nh
    epw = Eh // NW
    k = 200
    ks = 200
    assert Eh % NW == 0 and epw % k == 0 and k % 8 == 0 and epw % ks == 0

    f32 = jnp.float32
    bf16 = jnp.bfloat16
    fbits = jax.lax.bitcast_convert_type(
        x[:, POS:].astype(bf16), jnp.uint16).astype(jnp.uint32)
    packed = jax.lax.bitcast_convert_type(
        fbits[:, 0::2] | (fbits[:, 1::2] << 16), f32)
    xr = jnp.concatenate(
        [packed, x[:, :POS], jnp.zeros((N, DX - POS - F // 2), f32)], axis=1)

    w1cat = jnp.concatenate(
        [W1[:F:2], W1[F:2 * F:2], W1[1:F:2], W1[F + 1:2 * F:2]],
        axis=0).astype(bf16)
    w1r = W1[2 * F:2 * F + 1].astype(bf16)

    mesh = plsc.VectorSubcoreMesh(core_axis_name="c", subcore_axis_name="s")
    zrows = jnp.zeros((ZROWS, ACCW), f32)

    def gather_half(src, dst):
        return pl.kernel(
            lambda *refs: _sc_gather_body(epw, k, *refs),
            out_type=[jax.ShapeDtypeStruct((Eh, DX), f32),
                      jax.ShapeDtypeStruct((Eh, DX), f32)],
            mesh=mesh,
            scratch_types=[pltpu.VMEM((k,), jnp.int32),
                           pltpu.VMEM((k,), jnp.int32),
                           pltpu.VMEM((k, DX), f32),
                           pltpu.VMEM((k, DX), f32),
                           pltpu.SemaphoreType.DMA,
                           pltpu.SemaphoreType.DMA],
        )(src, dst, xr)

    def edge_mlp_half(gi, gj):
        return pl.pallas_call(
            _edge_mlp_kernel,
            grid=(Eh // EBLK,),
            in_specs=[
                pl.BlockSpec((EBLK, DX), lambda i: (i, 0)),
                pl.BlockSpec((EBLK, DX), lambda i: (i, 0)),
                pl.BlockSpec((2 * F, H1), lambda i: (0, 0)),
                pl.BlockSpec((1, H1), lambda i: (0, 0)),
                pl.BlockSpec((1, H1), lambda i: (0, 0)),
                pl.BlockSpec((H1, M), lambda i: (0, 0)),
                pl.BlockSpec((1, M), lambda i: (0, 0)),
                pl.BlockSpec((M, HC), lambda i: (0, 0)),
                pl.BlockSpec((1, HC), lambda i: (0, 0)),
                pl.BlockSpec((1, HC), lambda i: (0, 0)),
                pl.BlockSpec((1, 1), lambda i: (0, 0)),
                pl.BlockSpec((1, 1), lambda i: (0, 0)),
            ],
            out_specs=pl.BlockSpec((EBLK, ACCW), lambda i: (i, 0)),
            out_shape=jax.ShapeDtypeStruct((Eh, ACCW), f32),
        )(gi, gj, w1cat, w1r, b1[None].astype(bf16), W2.astype(bf16),
          b2[None], Wc1.astype(bf16), bc1[None],
          Wc2[:, 0][None], bc2[None], coors_scale[None])

    def scatter_half(dst, ev):
        return pl.kernel(
            lambda *refs: _sc_scatter_body(epw, ks, *refs),
            out_type=jax.ShapeDtypeStruct((NC, NPAD, ACCW), f32),
            mesh=mesh,
            scratch_types=[pltpu.VMEM((ks,), jnp.int32),
                           pltpu.VMEM((ks, ACCW), f32),
                           pltpu.VMEM_SHARED((NPAD, ACCW), f32)],
        )(dst, ev, zrows)

    srcs = [edge_index[0, h * Eh:(h + 1) * Eh] for h in range(nh)]
    dsts = [edge_index[1, h * Eh:(h + 1) * Eh] for h in range(nh)]
    gs = [gather_half(srcs[h], dsts[h]) for h in range(nh)]
    evs = [edge_mlp_half(gs[h][1], gs[h][0]) for h in range(nh)]
    ps = [scatter_half(dsts[h], evs[h]) for h in range(nh)]

    out = pl.pallas_call(
        _node_mlp_kernel,
        grid=(N // NBLK,),
        in_specs=[
            pl.BlockSpec((NBLK, POS + F), lambda i: (i, 0)),
            pl.BlockSpec((NBLK, ACCW), lambda i: (i, 0)),
            pl.BlockSpec((NBLK, ACCW), lambda i: (i, 0)),
            pl.BlockSpec((NBLK, ACCW), lambda i: (i, 0)),
            pl.BlockSpec((NBLK, ACCW), lambda i: (i, 0)),
            pl.BlockSpec((F, HN), lambda i: (0, 0)),
            pl.BlockSpec((M, HN), lambda i: (0, 0)),
            pl.BlockSpec((1, HN), lambda i: (0, 0)),
            pl.BlockSpec((HN, F), lambda i: (0, 0)),
            pl.BlockSpec((1, F), lambda i: (0, 0)),
        ],
        out_specs=pl.BlockSpec((NBLK, POS + F), lambda i: (i, 0)),
        out_shape=jax.ShapeDtypeStruct((N, POS + F), f32),
    )(x, ps[0][0, :N], ps[0][1, :N], ps[1][0, :N], ps[1][1, :N],
      Wn1[:F], Wn1[F:], bn1[None], Wn2, bn2[None])
    return out
